# baseline (speedup 1.0000x reference)
import functools

import jax
import jax.numpy as jnp
from jax.experimental import pallas as pl
from jax.experimental.pallas import tpu as pltpu
from jax.experimental.pallas import tpu_sc as plsc

_LAMBDA = 0.1
_D = 64
_H = 224
_W = 224
_TILE = 56
_NT = _H // _TILE
_S0, _C0 = 56, 96
_S1, _C1 = 28, 192
_HIGH = jax.lax.Precision.HIGHEST

_NPIX = 4 * _H * _W
_NW = 32
_CHUNK = _NPIX // _NW
_NV = _CHUNK // 16


def _sc_body(depths_hbm, bins_hbm, xv, bv):
    wid = jax.lax.axis_index("s") * 2 + jax.lax.axis_index("c")
    base = wid * _CHUNK
    pltpu.sync_copy(depths_hbm.at[pl.ds(base, _CHUNK)], xv)

    step = jnp.float32(15.625)
    inv = jnp.float32(1.0 / 15.625)

    def body(i, carry):
        x = xv[pl.ds(i * 16, 16)]
        b0f = (x * inv).astype(jnp.int32).astype(jnp.float32)
        up = 1.0 - jnp.maximum(-jnp.sign(x - (b0f + 1.0) * step), 0.0)
        dn = jnp.maximum(jnp.sign(b0f * step - x), 0.0)
        bi = jnp.minimum(jnp.maximum(b0f + up - dn, 0.0),
                         jnp.float32(_D - 1)).astype(jnp.int32)
        bv[pl.ds(i * 16, 16)] = bi
        return carry

    jax.lax.fori_loop(0, _NV, body, 0, unroll=8)
    pltpu.sync_copy(bv, bins_hbm.at[pl.ds(base, _CHUNK)])


_sc_bucketize = functools.partial(
    pl.kernel,
    out_type=jax.ShapeDtypeStruct((_NPIX,), jnp.int32),
    mesh=plsc.VectorSubcoreMesh(core_axis_name="c", subcore_axis_name="s"),
    scratch_types=[
        pltpu.VMEM((_CHUNK,), jnp.float32),
        pltpu.VMEM((_CHUNK,), jnp.int32),
    ],
)(_sc_body)


def _body(bins_ref, f0_ref, f1_ref, u0_ref, u1_ref,
          ut0_ref, ut1_ref, out_ref,
          wt0_ref, wt1_ref, h0hi_ref, h0lo_ref, h1hi_ref, h1lo_ref,
          sums0_ref, sums1_ref, cnt_ref):
    b = pl.program_id(0)
    t = pl.program_id(1)

    @pl.when((b == 0) & (t == 0))
    def _init():
        sums0_ref[...] = jnp.zeros_like(sums0_ref)
        sums1_ref[...] = jnp.zeros_like(sums1_ref)
        cnt_ref[...] = jnp.zeros_like(cnt_ref)

    bins = bins_ref[0]

    iota_d = jax.lax.broadcasted_iota(jnp.int32, (_D, _TILE, _W), 0)
    oh = (bins[None, :, :] == iota_d).astype(jnp.bfloat16)
    oh2 = oh.reshape(_D * _TILE, _W)

    def layer(u_ref, ut_ref, f_ref, wt_ref, hhi_ref, hlo_ref, s, c):
        w_t = jax.lax.dot_general(oh2, u_ref[...], (((1,), (0,)), ((), ())),
                                  preferred_element_type=jnp.float32)
        wt_ref[...] = w_t.reshape(_D, _TILE, s).astype(jnp.bfloat16)
        ht = jax.lax.dot_general(ut_ref[...], f_ref[0],
                                 (((1,), (0,)), ((), ())),
                                 precision=_HIGH)
        hhi = ht.astype(jnp.bfloat16)
        hhi_ref[...] = hhi
        hlo_ref[...] = (ht - hhi.astype(jnp.float32)).astype(jnp.bfloat16)
        w2 = wt_ref[...].reshape(_D, _TILE * s)
        return (jax.lax.dot_general(w2, hhi_ref[...].reshape(_TILE * s, c),
                                    (((1,), (0,)), ((), ())),
                                    preferred_element_type=jnp.float32)
                + jax.lax.dot_general(w2, hlo_ref[...].reshape(_TILE * s, c),
                                      (((1,), (0,)), ((), ())),
                                      preferred_element_type=jnp.float32))

    sums0_ref[...] = sums0_ref[...] + layer(
        u0_ref, ut0_ref, f0_ref, wt0_ref, h0hi_ref, h0lo_ref, _S0, _C0)
    sums1_ref[...] = sums1_ref[...] + layer(
        u1_ref, ut1_ref, f1_ref, wt1_ref, h1hi_ref, h1lo_ref, _S1, _C1)
    cnt_ref[...] = cnt_ref[...] + jnp.sum(
        wt0_ref[...].astype(jnp.float32), axis=(1, 2))[:, None]

    @pl.when((b == pl.num_programs(0) - 1) & (t == pl.num_programs(1) - 1))
    def _epilogue():
        cnt = cnt_ref[...]
        nonzero = cnt > 0.5
        denom = jnp.maximum(cnt, 1.0)

        def layer_term(sums, c):
            means = jnp.where(nonzero, sums / denom, 0.0)
            k = jax.lax.broadcasted_iota(jnp.int32, (_D, c), 1)
            d = jax.lax.broadcasted_iota(jnp.int32, (_D, c), 0)
            d_k = jnp.clip((k * 64) // c, 0, _D - 1)
            mask = (d == d_k).astype(jnp.float32)
            r_dk = jnp.sum(means * mask, axis=0, keepdims=True)
            sum_all = jnp.sum(means, axis=0, keepdims=True)
            r_rest = (sum_all - r_dk) / jnp.float32(_D - 1)
            aa = jnp.abs(r_dk)
            ab = jnp.abs(r_rest)
            s_k = (aa - ab) / (aa + ab + jnp.float32(1e-6))
            return jnp.sum(s_k) / jnp.float32(c)

        total = layer_term(sums0_ref[...], _C0) + layer_term(sums1_ref[...], _C1)
        val = jnp.float32(-_LAMBDA) * (total / jnp.float32(2.0))
        out_ref[...] = val.reshape(1, 1)


@functools.partial(jax.jit, static_argnums=())
def _run(depths_flat, f0r, f1r, u0b, u1b, u0, u1):
    bins_flat = _sc_bucketize(depths_flat)
    bins = bins_flat.reshape(4, _H, _W)
    out = pl.pallas_call(
        _body,
        grid=(4, _NT),
        in_specs=[
            pl.BlockSpec((1, _TILE, _W), lambda b, t: (b, t, 0)),
            pl.BlockSpec((1, _S0, _S0, _C0), lambda b, t: (b, 0, 0, 0)),
            pl.BlockSpec((1, _S1, _S1, _C1), lambda b, t: (b, 0, 0, 0)),
            pl.BlockSpec((_H, _S0), lambda b, t: (0, 0)),
            pl.BlockSpec((_H, _S1), lambda b, t: (0, 0)),
            pl.BlockSpec((_TILE, _S0), lambda b, t: (t, 0)),
            pl.BlockSpec((_TILE, _S1), lambda b, t: (t, 0)),
        ],
        out_specs=pl.BlockSpec((1, 1), lambda b, t: (0, 0)),
        out_shape=jax.ShapeDtypeStruct((1, 1), jnp.float32),
        scratch_shapes=[
            pltpu.VMEM((_D, _TILE, _S0), jnp.bfloat16),
            pltpu.VMEM((_D, _TILE, _S1), jnp.bfloat16),
            pltpu.VMEM((_TILE, _S0, _C0), jnp.bfloat16),
            pltpu.VMEM((_TILE, _S0, _C0), jnp.bfloat16),
            pltpu.VMEM((_TILE, _S1, _C1), jnp.bfloat16),
            pltpu.VMEM((_TILE, _S1, _C1), jnp.bfloat16),
            pltpu.VMEM((_D, _C0), jnp.float32),
            pltpu.VMEM((_D, _C1), jnp.float32),
            pltpu.VMEM((_D, 1), jnp.float32),
        ],
        compiler_params=pltpu.CompilerParams(
            dimension_semantics=("arbitrary", "arbitrary"),
        ),
    )(bins, f0r, f1r, u0b, u1b, u0, u1)
    return out.reshape(())


def kernel(imgs, depths, fmap0, fmap1):
    del imgs
    depths_flat = depths[:, 0, :, :].reshape(-1)
    u0 = jax.image.resize(jnp.eye(_S0, dtype=jnp.float32), (_H, _S0), "bilinear")
    u1 = jax.image.resize(jnp.eye(_S1, dtype=jnp.float32), (_H, _S1), "bilinear")
    u0b = u0.astype(jnp.bfloat16)
    u1b = u1.astype(jnp.bfloat16)
    f0r = jnp.transpose(fmap0, (0, 2, 3, 1))
    f1r = jnp.transpose(fmap1, (0, 2, 3, 1))
    return _run(depths_flat, f0r, f1r, u0b, u1b, u0, u1)


# speedup vs baseline: 1.1110x; 1.1110x over previous
import functools

import jax
import jax.numpy as jnp
from jax.experimental import pallas as pl
from jax.experimental.pallas import tpu as pltpu
from jax.experimental.pallas import tpu_sc as plsc

_LAMBDA = 0.1
_D = 64
_H = 224
_W = 224
_TILE = 112
_NT = _H // _TILE
_S0, _C0 = 56, 96
_S1, _C1 = 28, 192
_HIGH = jax.lax.Precision.HIGHEST

_NPIX = 4 * _H * _W
_NW = 32
_CHUNK = _NPIX // _NW
_NV = _CHUNK // 16


def _sc_body(depths_hbm, bins_hbm, xv, bv):
    wid = jax.lax.axis_index("s") * 2 + jax.lax.axis_index("c")
    base = wid * _CHUNK
    pltpu.sync_copy(depths_hbm.at[pl.ds(base, _CHUNK)], xv)

    step = jnp.float32(15.625)
    inv = jnp.float32(1.0 / 15.625)

    def body(i, carry):
        x = xv[pl.ds(i * 16, 16)]
        b0f = (x * inv).astype(jnp.int32).astype(jnp.float32)
        up = 1.0 - jnp.maximum(-jnp.sign(x - (b0f + 1.0) * step), 0.0)
        dn = jnp.maximum(jnp.sign(b0f * step - x), 0.0)
        bi = jnp.minimum(jnp.maximum(b0f + up - dn, 0.0),
                         jnp.float32(_D - 1)).astype(jnp.int32)
        bv[pl.ds(i * 16, 16)] = bi
        return carry

    jax.lax.fori_loop(0, _NV, body, 0)
    pltpu.sync_copy(bv, bins_hbm.at[pl.ds(base, _CHUNK)])


_sc_bucketize = functools.partial(
    pl.kernel,
    out_type=jax.ShapeDtypeStruct((_NPIX,), jnp.int32),
    mesh=plsc.VectorSubcoreMesh(core_axis_name="c", subcore_axis_name="s"),
    scratch_types=[
        pltpu.VMEM((_CHUNK,), jnp.float32),
        pltpu.VMEM((_CHUNK,), jnp.int32),
    ],
)(_sc_body)


def _body(bins_ref, f0_ref, f1_ref, u0_ref, u1_ref,
          ut0_ref, ut1_ref, out_ref,
          wt0_ref, wt1_ref, h0hi_ref, h0lo_ref, h1hi_ref, h1lo_ref,
          sums0_ref, sums1_ref, cnt_ref):
    b = pl.program_id(0)
    t = pl.program_id(1)

    @pl.when((b == 0) & (t == 0))
    def _init():
        sums0_ref[...] = jnp.zeros_like(sums0_ref)
        sums1_ref[...] = jnp.zeros_like(sums1_ref)
        cnt_ref[...] = jnp.zeros_like(cnt_ref)

    bins = bins_ref[0]

    iota_d = jax.lax.broadcasted_iota(jnp.int32, (_D, _TILE, _W), 0)
    oh = (bins[None, :, :] == iota_d).astype(jnp.bfloat16)
    oh2 = oh.reshape(_D * _TILE, _W)

    def layer(u_ref, ut_ref, f_ref, wt_ref, hhi_ref, hlo_ref, s, c):
        w_t = jax.lax.dot_general(oh2, u_ref[...], (((1,), (0,)), ((), ())),
                                  preferred_element_type=jnp.float32)
        wt_ref[...] = w_t.reshape(_D, _TILE, s).astype(jnp.bfloat16)
        ht = jax.lax.dot_general(ut_ref[...], f_ref[0],
                                 (((1,), (0,)), ((), ())),
                                 precision=_HIGH)
        hhi = ht.astype(jnp.bfloat16)
        hhi_ref[...] = hhi
        hlo_ref[...] = (ht - hhi.astype(jnp.float32)).astype(jnp.bfloat16)
        w2 = wt_ref[...].reshape(_D, _TILE * s)
        return (jax.lax.dot_general(w2, hhi_ref[...].reshape(_TILE * s, c),
                                    (((1,), (0,)), ((), ())),
                                    preferred_element_type=jnp.float32)
                + jax.lax.dot_general(w2, hlo_ref[...].reshape(_TILE * s, c),
                                      (((1,), (0,)), ((), ())),
                                      preferred_element_type=jnp.float32))

    sums0_ref[...] = sums0_ref[...] + layer(
        u0_ref, ut0_ref, f0_ref, wt0_ref, h0hi_ref, h0lo_ref, _S0, _C0)
    sums1_ref[...] = sums1_ref[...] + layer(
        u1_ref, ut1_ref, f1_ref, wt1_ref, h1hi_ref, h1lo_ref, _S1, _C1)
    cnt_ref[...] = cnt_ref[...] + jnp.sum(
        wt0_ref[...].astype(jnp.float32), axis=(1, 2))[:, None]

    @pl.when((b == pl.num_programs(0) - 1) & (t == pl.num_programs(1) - 1))
    def _epilogue():
        cnt = cnt_ref[...]
        nonzero = cnt > 0.5
        denom = jnp.maximum(cnt, 1.0)

        def layer_term(sums, c):
            means = jnp.where(nonzero, sums / denom, 0.0)
            k = jax.lax.broadcasted_iota(jnp.int32, (_D, c), 1)
            d = jax.lax.broadcasted_iota(jnp.int32, (_D, c), 0)
            d_k = jnp.clip((k * 64) // c, 0, _D - 1)
            mask = (d == d_k).astype(jnp.float32)
            r_dk = jnp.sum(means * mask, axis=0, keepdims=True)
            sum_all = jnp.sum(means, axis=0, keepdims=True)
            r_rest = (sum_all - r_dk) / jnp.float32(_D - 1)
            aa = jnp.abs(r_dk)
            ab = jnp.abs(r_rest)
            s_k = (aa - ab) / (aa + ab + jnp.float32(1e-6))
            return jnp.sum(s_k) / jnp.float32(c)

        total = layer_term(sums0_ref[...], _C0) + layer_term(sums1_ref[...], _C1)
        val = jnp.float32(-_LAMBDA) * (total / jnp.float32(2.0))
        out_ref[...] = val.reshape(1, 1)


@functools.partial(jax.jit, static_argnums=())
def _run(depths_flat, f0r, f1r, u0b, u1b, u0, u1):
    bins_flat = _sc_bucketize(depths_flat)
    bins = bins_flat.reshape(4, _H, _W)
    out = pl.pallas_call(
        _body,
        grid=(4, _NT),
        in_specs=[
            pl.BlockSpec((1, _TILE, _W), lambda b, t: (b, t, 0)),
            pl.BlockSpec((1, _S0, _S0, _C0), lambda b, t: (b, 0, 0, 0)),
            pl.BlockSpec((1, _S1, _S1, _C1), lambda b, t: (b, 0, 0, 0)),
            pl.BlockSpec((_H, _S0), lambda b, t: (0, 0)),
            pl.BlockSpec((_H, _S1), lambda b, t: (0, 0)),
            pl.BlockSpec((_TILE, _S0), lambda b, t: (t, 0)),
            pl.BlockSpec((_TILE, _S1), lambda b, t: (t, 0)),
        ],
        out_specs=pl.BlockSpec((1, 1), lambda b, t: (0, 0)),
        out_shape=jax.ShapeDtypeStruct((1, 1), jnp.float32),
        scratch_shapes=[
            pltpu.VMEM((_D, _TILE, _S0), jnp.bfloat16),
            pltpu.VMEM((_D, _TILE, _S1), jnp.bfloat16),
            pltpu.VMEM((_TILE, _S0, _C0), jnp.bfloat16),
            pltpu.VMEM((_TILE, _S0, _C0), jnp.bfloat16),
            pltpu.VMEM((_TILE, _S1, _C1), jnp.bfloat16),
            pltpu.VMEM((_TILE, _S1, _C1), jnp.bfloat16),
            pltpu.VMEM((_D, _C0), jnp.float32),
            pltpu.VMEM((_D, _C1), jnp.float32),
            pltpu.VMEM((_D, 1), jnp.float32),
        ],
        compiler_params=pltpu.CompilerParams(
            dimension_semantics=("arbitrary", "arbitrary"),
        ),
    )(bins, f0r, f1r, u0b, u1b, u0, u1)
    return out.reshape(())


def kernel(imgs, depths, fmap0, fmap1):
    del imgs
    depths_flat = depths[:, 0, :, :].reshape(-1)
    u0 = jax.image.resize(jnp.eye(_S0, dtype=jnp.float32), (_H, _S0), "bilinear")
    u1 = jax.image.resize(jnp.eye(_S1, dtype=jnp.float32), (_H, _S1), "bilinear")
    u0b = u0.astype(jnp.bfloat16)
    u1b = u1.astype(jnp.bfloat16)
    f0r = jnp.transpose(fmap0, (0, 2, 3, 1))
    f1r = jnp.transpose(fmap1, (0, 2, 3, 1))
    return _run(depths_flat, f0r, f1r, u0b, u1b, u0, u1)


# speedup vs baseline: 1.1319x; 1.0188x over previous
import functools

import jax
import jax.numpy as jnp
from jax.experimental import pallas as pl
from jax.experimental.pallas import tpu as pltpu
from jax.experimental.pallas import tpu_sc as plsc

_LAMBDA = 0.1
_D = 64
_H = 224
_W = 224
_TILE = 224
_NT = _H // _TILE
_S0, _C0 = 56, 96
_S1, _C1 = 28, 192
_HIGH = jax.lax.Precision.HIGHEST

_NPIX = 4 * _H * _W
_NW = 32
_CHUNK = _NPIX // _NW
_NV = _CHUNK // 16


def _sc_body(depths_hbm, bins_hbm, xv, bv):
    wid = jax.lax.axis_index("s") * 2 + jax.lax.axis_index("c")
    base = wid * _CHUNK
    pltpu.sync_copy(depths_hbm.at[pl.ds(base, _CHUNK)], xv)

    step = jnp.float32(15.625)
    inv = jnp.float32(1.0 / 15.625)

    def body(i, carry):
        x = xv[pl.ds(i * 16, 16)]
        b0f = (x * inv).astype(jnp.int32).astype(jnp.float32)
        up = 1.0 - jnp.maximum(-jnp.sign(x - (b0f + 1.0) * step), 0.0)
        dn = jnp.maximum(jnp.sign(b0f * step - x), 0.0)
        bi = jnp.minimum(jnp.maximum(b0f + up - dn, 0.0),
                         jnp.float32(_D - 1)).astype(jnp.int32)
        bv[pl.ds(i * 16, 16)] = bi
        return carry

    jax.lax.fori_loop(0, _NV, body, 0)
    pltpu.sync_copy(bv, bins_hbm.at[pl.ds(base, _CHUNK)])


_sc_bucketize = functools.partial(
    pl.kernel,
    out_type=jax.ShapeDtypeStruct((_NPIX,), jnp.int32),
    mesh=plsc.VectorSubcoreMesh(core_axis_name="c", subcore_axis_name="s"),
    scratch_types=[
        pltpu.VMEM((_CHUNK,), jnp.float32),
        pltpu.VMEM((_CHUNK,), jnp.int32),
    ],
)(_sc_body)


def _body(bins_ref, f0_ref, f1_ref, u0_ref, u1_ref,
          ut0_ref, ut1_ref, out_ref,
          wt0_ref, wt1_ref, h0hi_ref, h0lo_ref, h1hi_ref, h1lo_ref,
          sums0_ref, sums1_ref, cnt_ref):
    b = pl.program_id(0)
    t = pl.program_id(1)

    @pl.when((b == 0) & (t == 0))
    def _init():
        sums0_ref[...] = jnp.zeros_like(sums0_ref)
        sums1_ref[...] = jnp.zeros_like(sums1_ref)
        cnt_ref[...] = jnp.zeros_like(cnt_ref)

    bins = bins_ref[0]

    iota_d = jax.lax.broadcasted_iota(jnp.int32, (_D, _TILE, _W), 0)
    oh = (bins[None, :, :] == iota_d).astype(jnp.bfloat16)
    oh2 = oh.reshape(_D * _TILE, _W)

    def layer(u_ref, ut_ref, f_ref, wt_ref, hhi_ref, hlo_ref, s, c):
        w_t = jax.lax.dot_general(oh2, u_ref[...], (((1,), (0,)), ((), ())),
                                  preferred_element_type=jnp.float32)
        wt_ref[...] = w_t.reshape(_D, _TILE, s).astype(jnp.bfloat16)
        ht = jax.lax.dot_general(ut_ref[...], f_ref[0],
                                 (((1,), (0,)), ((), ())),
                                 precision=_HIGH)
        hhi = ht.astype(jnp.bfloat16)
        hhi_ref[...] = hhi
        hlo_ref[...] = (ht - hhi.astype(jnp.float32)).astype(jnp.bfloat16)
        w2 = wt_ref[...].reshape(_D, _TILE * s)
        return (jax.lax.dot_general(w2, hhi_ref[...].reshape(_TILE * s, c),
                                    (((1,), (0,)), ((), ())),
                                    preferred_element_type=jnp.float32)
                + jax.lax.dot_general(w2, hlo_ref[...].reshape(_TILE * s, c),
                                      (((1,), (0,)), ((), ())),
                                      preferred_element_type=jnp.float32))

    sums0_ref[...] = sums0_ref[...] + layer(
        u0_ref, ut0_ref, f0_ref, wt0_ref, h0hi_ref, h0lo_ref, _S0, _C0)
    sums1_ref[...] = sums1_ref[...] + layer(
        u1_ref, ut1_ref, f1_ref, wt1_ref, h1hi_ref, h1lo_ref, _S1, _C1)
    cnt_ref[...] = cnt_ref[...] + jnp.sum(
        wt0_ref[...].astype(jnp.float32), axis=(1, 2))[:, None]

    @pl.when((b == pl.num_programs(0) - 1) & (t == pl.num_programs(1) - 1))
    def _epilogue():
        cnt = cnt_ref[...]
        nonzero = cnt > 0.5
        denom = jnp.maximum(cnt, 1.0)

        def layer_term(sums, c):
            means = jnp.where(nonzero, sums / denom, 0.0)
            k = jax.lax.broadcasted_iota(jnp.int32, (_D, c), 1)
            d = jax.lax.broadcasted_iota(jnp.int32, (_D, c), 0)
            d_k = jnp.clip((k * 64) // c, 0, _D - 1)
            mask = (d == d_k).astype(jnp.float32)
            r_dk = jnp.sum(means * mask, axis=0, keepdims=True)
            sum_all = jnp.sum(means, axis=0, keepdims=True)
            r_rest = (sum_all - r_dk) / jnp.float32(_D - 1)
            aa = jnp.abs(r_dk)
            ab = jnp.abs(r_rest)
            s_k = (aa - ab) / (aa + ab + jnp.float32(1e-6))
            return jnp.sum(s_k) / jnp.float32(c)

        total = layer_term(sums0_ref[...], _C0) + layer_term(sums1_ref[...], _C1)
        val = jnp.float32(-_LAMBDA) * (total / jnp.float32(2.0))
        out_ref[...] = val.reshape(1, 1)


@functools.partial(jax.jit, static_argnums=())
def _run(depths_flat, f0r, f1r, u0b, u1b, u0, u1):
    bins_flat = _sc_bucketize(depths_flat)
    bins = bins_flat.reshape(4, _H, _W)
    out = pl.pallas_call(
        _body,
        grid=(4, _NT),
        in_specs=[
            pl.BlockSpec((1, _TILE, _W), lambda b, t: (b, t, 0)),
            pl.BlockSpec((1, _S0, _S0, _C0), lambda b, t: (b, 0, 0, 0)),
            pl.BlockSpec((1, _S1, _S1, _C1), lambda b, t: (b, 0, 0, 0)),
            pl.BlockSpec((_H, _S0), lambda b, t: (0, 0)),
            pl.BlockSpec((_H, _S1), lambda b, t: (0, 0)),
            pl.BlockSpec((_TILE, _S0), lambda b, t: (t, 0)),
            pl.BlockSpec((_TILE, _S1), lambda b, t: (t, 0)),
        ],
        out_specs=pl.BlockSpec((1, 1), lambda b, t: (0, 0)),
        out_shape=jax.ShapeDtypeStruct((1, 1), jnp.float32),
        scratch_shapes=[
            pltpu.VMEM((_D, _TILE, _S0), jnp.bfloat16),
            pltpu.VMEM((_D, _TILE, _S1), jnp.bfloat16),
            pltpu.VMEM((_TILE, _S0, _C0), jnp.bfloat16),
            pltpu.VMEM((_TILE, _S0, _C0), jnp.bfloat16),
            pltpu.VMEM((_TILE, _S1, _C1), jnp.bfloat16),
            pltpu.VMEM((_TILE, _S1, _C1), jnp.bfloat16),
            pltpu.VMEM((_D, _C0), jnp.float32),
            pltpu.VMEM((_D, _C1), jnp.float32),
            pltpu.VMEM((_D, 1), jnp.float32),
        ],
        compiler_params=pltpu.CompilerParams(
            dimension_semantics=("arbitrary", "arbitrary"),
        ),
    )(bins, f0r, f1r, u0b, u1b, u0, u1)
    return out.reshape(())


def kernel(imgs, depths, fmap0, fmap1):
    del imgs
    depths_flat = depths[:, 0, :, :].reshape(-1)
    u0 = jax.image.resize(jnp.eye(_S0, dtype=jnp.float32), (_H, _S0), "bilinear")
    u1 = jax.image.resize(jnp.eye(_S1, dtype=jnp.float32), (_H, _S1), "bilinear")
    u0b = u0.astype(jnp.bfloat16)
    u1b = u1.astype(jnp.bfloat16)
    f0r = jnp.transpose(fmap0, (0, 2, 3, 1))
    f1r = jnp.transpose(fmap1, (0, 2, 3, 1))
    return _run(depths_flat, f0r, f1r, u0b, u1b, u0, u1)
